# baseline (speedup 1.0000x reference)
import functools

import jax
import jax.numpy as jnp
from jax import lax
from jax.experimental import pallas as pl
from jax.experimental.pallas import tpu as pltpu
from jax.experimental.pallas import tpu_sc as plsc

POS_IOU_THR = 0.5
NEG_IOU_THR = 0.4

NUM_CORES = 2
NUM_SUBCORES = 16
LANES = 16
NW = NUM_CORES * NUM_SUBCORES


CHUNKS_PER_STEP = 4
GT_UNROLL = 2


@functools.lru_cache(maxsize=None)
def _build_assign(sc_n: int, num_gt: int):
    per_w = sc_n // NW
    chunks = per_w // LANES
    assert per_w % 128 == 0 and chunks % CHUNKS_PER_STEP == 0
    gstride = num_gt * LANES
    mesh = plsc.VectorSubcoreMesh(core_axis_name="c", subcore_axis_name="s",
                                  num_cores=NUM_CORES)

    @functools.partial(
        pl.kernel,
        mesh=mesh,
        out_type=(
            jax.ShapeDtypeStruct((sc_n,), jnp.int32),
            jax.ShapeDtypeStruct((sc_n,), jnp.float32),
            jax.ShapeDtypeStruct((sc_n,), jnp.int32),
        ),
        scratch_types=[
            pltpu.VMEM((4, per_w), jnp.float32),
            pltpu.VMEM((5 * gstride,), jnp.float32),
            pltpu.VMEM((gstride,), jnp.int32),
            pltpu.VMEM((per_w,), jnp.int32),
            pltpu.VMEM((per_w,), jnp.float32),
            pltpu.VMEM((per_w,), jnp.int32),
            pltpu.VMEM((8 * (num_gt // 8 + 1) * 8,), jnp.float32),
        ],
    )
    def assign(coords_hbm, tgt_hbm,
               inds_hbm, maxov_hbm, albl_hbm,
               coords_v, gt_v, lbl_v, inds_v, maxov_v, albl_v, tgt_v):
        wid = lax.axis_index("s") * NUM_CORES + lax.axis_index("c")
        base = wid * per_w
        pltpu.sync_copy(coords_hbm.at[:, pl.ds(base, per_w)], coords_v)
        pltpu.sync_copy(tgt_hbm, tgt_v)

        def build_body(j, carry):
            row = tgt_v[pl.ds(j * 8, LANES)]
            lab = row[4]
            k = lab != -1.0
            gx1 = jnp.where(k, row[0], 0.0)
            gy1 = jnp.where(k, row[1], 0.0)
            gx2 = jnp.where(k, row[2], 0.0)
            gy2 = jnp.where(k, row[3], 0.0)
            ga = (jnp.maximum(gx2 - gx1, 0.0) *
                  jnp.maximum(gy2 - gy1, 0.0))
            j16 = j * LANES
            gt_v[pl.ds(j16, LANES)] = jnp.full((LANES,), gx1)
            gt_v[pl.ds(j16 + gstride, LANES)] = jnp.full((LANES,), gy1)
            gt_v[pl.ds(j16 + 2 * gstride, LANES)] = jnp.full((LANES,), gx2)
            gt_v[pl.ds(j16 + 3 * gstride, LANES)] = jnp.full((LANES,), gy2)
            gt_v[pl.ds(j16 + 4 * gstride, LANES)] = jnp.full((LANES,), ga)
            lbl_v[pl.ds(j16, LANES)] = jnp.full(
                (LANES,), lab).astype(jnp.int32)
            return carry

        lax.fori_loop(0, num_gt, build_body, 0, unroll=4)

        nC = CHUNKS_PER_STEP

        def chunk_body(c, carry):
            offs = [c * (nC * LANES) + k * LANES for k in range(nC)]
            ax1 = [coords_v[0, pl.ds(o, LANES)] for o in offs]
            ay1 = [coords_v[1, pl.ds(o, LANES)] for o in offs]
            ax2 = [coords_v[2, pl.ds(o, LANES)] for o in offs]
            ay2 = [coords_v[3, pl.ds(o, LANES)] for o in offs]
            area1 = [(jnp.maximum(ax2[k] - ax1[k], 0.0) *
                      jnp.maximum(ay2[k] - ay1[k], 0.0)) for k in range(nC)]

            def gt_body(j, bcarry):
                best, bestj, bestl = bcarry
                j16 = j * LANES
                gx1 = gt_v[pl.ds(j16, LANES)]
                gy1 = gt_v[pl.ds(j16 + gstride, LANES)]
                gx2 = gt_v[pl.ds(j16 + 2 * gstride, LANES)]
                gy2 = gt_v[pl.ds(j16 + 3 * gstride, LANES)]
                ga = gt_v[pl.ds(j16 + 4 * gstride, LANES)]
                glb = lbl_v[pl.ds(j16, LANES)]
                jvec = jnp.full((LANES,), j, jnp.int32)
                best, bestj, bestl = list(best), list(bestj), list(bestl)
                for k in range(nC):
                    iw = jnp.maximum(
                        jnp.minimum(ax2[k], gx2) - jnp.maximum(ax1[k], gx1),
                        0.0)
                    ih = jnp.maximum(
                        jnp.minimum(ay2[k], gy2) - jnp.maximum(ay1[k], gy1),
                        0.0)
                    inter = iw * ih
                    union = (area1[k] + ga) - inter
                    iou = inter / jnp.maximum(union, 1e-10)
                    upd = iou > best[k]
                    best[k] = jnp.where(upd, iou, best[k])
                    bestj[k] = jnp.where(upd, jvec, bestj[k])
                    bestl[k] = jnp.where(upd, glb, bestl[k])
                return tuple(best), tuple(bestj), tuple(bestl)

            best0 = tuple(jnp.full((LANES,), -1.0, jnp.float32)
                          for _ in range(nC))
            bestj0 = tuple(jnp.zeros((LANES,), jnp.int32) for _ in range(nC))
            best, bestj, bestl = lax.fori_loop(
                0, num_gt, gt_body, (best0, bestj0, bestj0),
                unroll=GT_UNROLL)

            for k in range(nC):
                pos = best[k] > POS_IOU_THR
                neg = best[k] < NEG_IOU_THR
                inds = jnp.where(
                    pos, bestj[k] + 1,
                    jnp.where(neg, jnp.full((LANES,), 0, jnp.int32),
                              jnp.full((LANES,), -1, jnp.int32)))
                albl = jnp.where(pos, bestl[k],
                                 jnp.full((LANES,), -1, jnp.int32))
                inds_v[pl.ds(offs[k], LANES)] = inds
                maxov_v[pl.ds(offs[k], LANES)] = best[k]
                albl_v[pl.ds(offs[k], LANES)] = albl
            return carry

        lax.fori_loop(0, chunks // nC, chunk_body, 0)
        pltpu.sync_copy(inds_v, inds_hbm.at[pl.ds(base, per_w)])
        pltpu.sync_copy(maxov_v, maxov_hbm.at[pl.ds(base, per_w)])
        pltpu.sync_copy(albl_v, albl_hbm.at[pl.ds(base, per_w)])

    return assign


TC_BLOCK = 2048
SC_N = 4096


@functools.lru_cache(maxsize=None)
def _build_tc_assign(num_anchors: int, sc_n: int, num_gt_pad: int):
    tc_real = num_anchors - sc_n
    nblk = (tc_real + TC_BLOCK - 1) // TC_BLOCK
    col0 = sc_n // TC_BLOCK

    def body(coords_ref, tgt_ref, inds_ref, maxov_ref, albl_ref):
        ax1 = coords_ref[0:1, :]
        ay1 = coords_ref[1:2, :]
        ax2 = coords_ref[2:3, :]
        ay2 = coords_ref[3:4, :]
        area1 = (jnp.maximum(ax2 - ax1, 0.0) *
                 jnp.maximum(ay2 - ay1, 0.0))
        tlab = tgt_ref[:, 4:5]
        keep = tlab != -1.0
        gx1 = jnp.where(keep, tgt_ref[:, 0:1], 0.0)
        gy1 = jnp.where(keep, tgt_ref[:, 1:2], 0.0)
        gx2 = jnp.where(keep, tgt_ref[:, 2:3], 0.0)
        gy2 = jnp.where(keep, tgt_ref[:, 3:4], 0.0)
        ga = (jnp.maximum(gx2 - gx1, 0.0) *
              jnp.maximum(gy2 - gy1, 0.0))
        lbl = tlab.astype(jnp.int32)
        iw = jnp.maximum(jnp.minimum(ax2, gx2) - jnp.maximum(ax1, gx1), 0.0)
        ih = jnp.maximum(jnp.minimum(ay2, gy2) - jnp.maximum(ay1, gy1), 0.0)
        inter = iw * ih
        union = (area1 + ga) - inter
        iou = inter / jnp.maximum(union, 1e-10)
        mx = jnp.max(iou, axis=0, keepdims=True)
        sub = lax.broadcasted_iota(jnp.int32, iou.shape, 0) << 16
        key = sub | ((lbl + 32768) & 0xFFFF)
        kmin = jnp.min(jnp.where(iou == mx, key, jnp.int32(0x7FFFFFFF)),
                       axis=0, keepdims=True)
        am = kmin >> 16
        lblv = (kmin & 0xFFFF) - 32768
        pos = mx > POS_IOU_THR
        neg = mx < NEG_IOU_THR
        inds_ref[...] = jnp.where(
            pos, am + 1,
            jnp.where(neg, jnp.int32(0), jnp.int32(-1)))[0]
        maxov_ref[...] = mx[0]
        albl_ref[...] = jnp.where(pos, lblv, jnp.int32(-1))[0]

    return pl.pallas_call(
        body,
        grid=(nblk,),
        in_specs=[
            pl.BlockSpec((4, TC_BLOCK), lambda i: (0, col0 + i)),
            pl.BlockSpec((num_gt_pad, 8), lambda i: (0, 0)),
        ],
        out_specs=[
            pl.BlockSpec((TC_BLOCK,), lambda i: (i,)),
            pl.BlockSpec((TC_BLOCK,), lambda i: (i,)),
            pl.BlockSpec((TC_BLOCK,), lambda i: (i,)),
        ],
        out_shape=[
            jax.ShapeDtypeStruct((tc_real,), jnp.int32),
            jax.ShapeDtypeStruct((tc_real,), jnp.float32),
            jax.ShapeDtypeStruct((tc_real,), jnp.int32),
        ],
    )


def kernel(bboxes, targets):
    num_anchors = bboxes.shape[0]
    num_gt = targets.shape[0]
    if num_gt == 0:
        return None

    num_gt_pad = (num_gt // 8 + 1) * 8
    tgt_pad = jnp.pad(targets, ((0, num_gt_pad - num_gt), (0, 3)))

    sc_n = min(SC_N, (num_anchors // 4096) * 4096)
    coords = bboxes.T

    parts = []
    if num_anchors - sc_n > 0:
        assign_tc = _build_tc_assign(num_anchors, sc_n, num_gt_pad)
        parts.append(assign_tc(coords, tgt_pad))
    if sc_n > 0:
        assign_sc = _build_assign(sc_n, num_gt)
        parts.insert(0, assign_sc(coords, tgt_pad.reshape(-1)))
    if len(parts) == 1:
        return tuple(parts[0])
    return tuple(jnp.concatenate([a, b]) for a, b in zip(*parts))


# speedup vs baseline: 1.0117x; 1.0117x over previous
import functools

import jax
import jax.numpy as jnp
from jax import lax
from jax.experimental import pallas as pl
from jax.experimental.pallas import tpu as pltpu
from jax.experimental.pallas import tpu_sc as plsc

POS_IOU_THR = 0.5
NEG_IOU_THR = 0.4

NUM_CORES = 2
NUM_SUBCORES = 16
LANES = 16
NW = NUM_CORES * NUM_SUBCORES


CHUNKS_PER_STEP = 2
GT_UNROLL = 4


@functools.lru_cache(maxsize=None)
def _build_assign(sc_n: int, num_gt: int):
    per_w = sc_n // NW
    chunks = per_w // LANES
    assert per_w % 128 == 0 and chunks % CHUNKS_PER_STEP == 0
    gstride = num_gt * LANES
    mesh = plsc.VectorSubcoreMesh(core_axis_name="c", subcore_axis_name="s",
                                  num_cores=NUM_CORES)

    @functools.partial(
        pl.kernel,
        mesh=mesh,
        out_type=(
            jax.ShapeDtypeStruct((sc_n,), jnp.int32),
            jax.ShapeDtypeStruct((sc_n,), jnp.float32),
            jax.ShapeDtypeStruct((sc_n,), jnp.int32),
        ),
        scratch_types=[
            pltpu.VMEM((4, per_w), jnp.float32),
            pltpu.VMEM((5 * gstride,), jnp.float32),
            pltpu.VMEM((gstride,), jnp.int32),
            pltpu.VMEM((per_w,), jnp.int32),
            pltpu.VMEM((per_w,), jnp.float32),
            pltpu.VMEM((per_w,), jnp.int32),
            pltpu.VMEM((8 * (num_gt // 8 + 1) * 8,), jnp.float32),
        ],
    )
    def assign(coords_hbm, tgt_hbm,
               inds_hbm, maxov_hbm, albl_hbm,
               coords_v, gt_v, lbl_v, inds_v, maxov_v, albl_v, tgt_v):
        wid = lax.axis_index("s") * NUM_CORES + lax.axis_index("c")
        base = wid * per_w
        pltpu.sync_copy(coords_hbm.at[:, pl.ds(base, per_w)], coords_v)
        pltpu.sync_copy(tgt_hbm, tgt_v)

        def build_body(j, carry):
            row = tgt_v[pl.ds(j * 8, LANES)]
            lab = row[4]
            k = lab != -1.0
            gx1 = jnp.where(k, row[0], 0.0)
            gy1 = jnp.where(k, row[1], 0.0)
            gx2 = jnp.where(k, row[2], 0.0)
            gy2 = jnp.where(k, row[3], 0.0)
            ga = (jnp.maximum(gx2 - gx1, 0.0) *
                  jnp.maximum(gy2 - gy1, 0.0))
            j16 = j * LANES
            gt_v[pl.ds(j16, LANES)] = jnp.full((LANES,), gx1)
            gt_v[pl.ds(j16 + gstride, LANES)] = jnp.full((LANES,), gy1)
            gt_v[pl.ds(j16 + 2 * gstride, LANES)] = jnp.full((LANES,), gx2)
            gt_v[pl.ds(j16 + 3 * gstride, LANES)] = jnp.full((LANES,), gy2)
            gt_v[pl.ds(j16 + 4 * gstride, LANES)] = jnp.full((LANES,), ga)
            lbl_v[pl.ds(j16, LANES)] = jnp.full(
                (LANES,), lab).astype(jnp.int32)
            return carry

        lax.fori_loop(0, num_gt, build_body, 0)

        nC = CHUNKS_PER_STEP

        def chunk_body(c, carry):
            offs = [c * (nC * LANES) + k * LANES for k in range(nC)]
            ax1 = [coords_v[0, pl.ds(o, LANES)] for o in offs]
            ay1 = [coords_v[1, pl.ds(o, LANES)] for o in offs]
            ax2 = [coords_v[2, pl.ds(o, LANES)] for o in offs]
            ay2 = [coords_v[3, pl.ds(o, LANES)] for o in offs]
            area1 = [(jnp.maximum(ax2[k] - ax1[k], 0.0) *
                      jnp.maximum(ay2[k] - ay1[k], 0.0)) for k in range(nC)]

            def gt_body(j, bcarry):
                best, bestj, bestl = bcarry
                j16 = j * LANES
                gx1 = gt_v[pl.ds(j16, LANES)]
                gy1 = gt_v[pl.ds(j16 + gstride, LANES)]
                gx2 = gt_v[pl.ds(j16 + 2 * gstride, LANES)]
                gy2 = gt_v[pl.ds(j16 + 3 * gstride, LANES)]
                ga = gt_v[pl.ds(j16 + 4 * gstride, LANES)]
                glb = lbl_v[pl.ds(j16, LANES)]
                jvec = jnp.full((LANES,), j, jnp.int32)
                best, bestj, bestl = list(best), list(bestj), list(bestl)
                for k in range(nC):
                    iw = jnp.maximum(
                        jnp.minimum(ax2[k], gx2) - jnp.maximum(ax1[k], gx1),
                        0.0)
                    ih = jnp.maximum(
                        jnp.minimum(ay2[k], gy2) - jnp.maximum(ay1[k], gy1),
                        0.0)
                    inter = iw * ih
                    union = (area1[k] + ga) - inter
                    iou = inter / jnp.maximum(union, 1e-10)
                    upd = iou > best[k]
                    best[k] = jnp.where(upd, iou, best[k])
                    bestj[k] = jnp.where(upd, jvec, bestj[k])
                    bestl[k] = jnp.where(upd, glb, bestl[k])
                return tuple(best), tuple(bestj), tuple(bestl)

            best0 = tuple(jnp.full((LANES,), -1.0, jnp.float32)
                          for _ in range(nC))
            bestj0 = tuple(jnp.zeros((LANES,), jnp.int32) for _ in range(nC))
            best, bestj, bestl = lax.fori_loop(
                0, num_gt, gt_body, (best0, bestj0, bestj0),
                unroll=GT_UNROLL)

            for k in range(nC):
                pos = best[k] > POS_IOU_THR
                neg = best[k] < NEG_IOU_THR
                inds = jnp.where(
                    pos, bestj[k] + 1,
                    jnp.where(neg, jnp.full((LANES,), 0, jnp.int32),
                              jnp.full((LANES,), -1, jnp.int32)))
                albl = jnp.where(pos, bestl[k],
                                 jnp.full((LANES,), -1, jnp.int32))
                inds_v[pl.ds(offs[k], LANES)] = inds
                maxov_v[pl.ds(offs[k], LANES)] = best[k]
                albl_v[pl.ds(offs[k], LANES)] = albl
            return carry

        lax.fori_loop(0, chunks // nC, chunk_body, 0)
        pltpu.sync_copy(inds_v, inds_hbm.at[pl.ds(base, per_w)])
        pltpu.sync_copy(maxov_v, maxov_hbm.at[pl.ds(base, per_w)])
        pltpu.sync_copy(albl_v, albl_hbm.at[pl.ds(base, per_w)])

    return assign


TC_BLOCK = 2048
SC_N = 4096


@functools.lru_cache(maxsize=None)
def _build_tc_assign(num_anchors: int, sc_n: int, num_gt_pad: int):
    tc_real = num_anchors - sc_n
    nblk = (tc_real + TC_BLOCK - 1) // TC_BLOCK
    col0 = sc_n // TC_BLOCK

    def body(coords_ref, tgt_ref, inds_ref, maxov_ref, albl_ref):
        ax1 = coords_ref[0:1, :]
        ay1 = coords_ref[1:2, :]
        ax2 = coords_ref[2:3, :]
        ay2 = coords_ref[3:4, :]
        area1 = (jnp.maximum(ax2 - ax1, 0.0) *
                 jnp.maximum(ay2 - ay1, 0.0))
        tlab = tgt_ref[:, 4:5]
        keep = tlab != -1.0
        gx1 = jnp.where(keep, tgt_ref[:, 0:1], 0.0)
        gy1 = jnp.where(keep, tgt_ref[:, 1:2], 0.0)
        gx2 = jnp.where(keep, tgt_ref[:, 2:3], 0.0)
        gy2 = jnp.where(keep, tgt_ref[:, 3:4], 0.0)
        ga = (jnp.maximum(gx2 - gx1, 0.0) *
              jnp.maximum(gy2 - gy1, 0.0))
        lbl = tlab.astype(jnp.int32)
        iw = jnp.maximum(jnp.minimum(ax2, gx2) - jnp.maximum(ax1, gx1), 0.0)
        ih = jnp.maximum(jnp.minimum(ay2, gy2) - jnp.maximum(ay1, gy1), 0.0)
        inter = iw * ih
        union = (area1 + ga) - inter
        iou = inter / jnp.maximum(union, 1e-10)
        mx = jnp.max(iou, axis=0, keepdims=True)
        sub = lax.broadcasted_iota(jnp.int32, iou.shape, 0) << 16
        key = sub | ((lbl + 32768) & 0xFFFF)
        kmin = jnp.min(jnp.where(iou == mx, key, jnp.int32(0x7FFFFFFF)),
                       axis=0, keepdims=True)
        am = kmin >> 16
        lblv = (kmin & 0xFFFF) - 32768
        pos = mx > POS_IOU_THR
        neg = mx < NEG_IOU_THR
        inds_ref[...] = jnp.where(
            pos, am + 1,
            jnp.where(neg, jnp.int32(0), jnp.int32(-1)))[0]
        maxov_ref[...] = mx[0]
        albl_ref[...] = jnp.where(pos, lblv, jnp.int32(-1))[0]

    return pl.pallas_call(
        body,
        grid=(nblk,),
        in_specs=[
            pl.BlockSpec((4, TC_BLOCK), lambda i: (0, col0 + i)),
            pl.BlockSpec((num_gt_pad, 8), lambda i: (0, 0)),
        ],
        out_specs=[
            pl.BlockSpec((TC_BLOCK,), lambda i: (i,)),
            pl.BlockSpec((TC_BLOCK,), lambda i: (i,)),
            pl.BlockSpec((TC_BLOCK,), lambda i: (i,)),
        ],
        out_shape=[
            jax.ShapeDtypeStruct((tc_real,), jnp.int32),
            jax.ShapeDtypeStruct((tc_real,), jnp.float32),
            jax.ShapeDtypeStruct((tc_real,), jnp.int32),
        ],
    )


def kernel(bboxes, targets):
    num_anchors = bboxes.shape[0]
    num_gt = targets.shape[0]
    if num_gt == 0:
        return None

    num_gt_pad = (num_gt // 8 + 1) * 8
    tgt_pad = jnp.pad(targets, ((0, num_gt_pad - num_gt), (0, 3)))

    sc_n = min(SC_N, (num_anchors // 4096) * 4096)
    coords = bboxes.T

    parts = []
    if num_anchors - sc_n > 0:
        assign_tc = _build_tc_assign(num_anchors, sc_n, num_gt_pad)
        parts.append(assign_tc(coords, tgt_pad))
    if sc_n > 0:
        assign_sc = _build_assign(sc_n, num_gt)
        parts.insert(0, assign_sc(coords, tgt_pad.reshape(-1)))
    if len(parts) == 1:
        return tuple(parts[0])
    return tuple(jnp.concatenate([a, b]) for a, b in zip(*parts))


# speedup vs baseline: 1.0164x; 1.0046x over previous
import functools

import jax
import jax.numpy as jnp
from jax import lax
from jax.experimental import pallas as pl
from jax.experimental.pallas import tpu as pltpu
from jax.experimental.pallas import tpu_sc as plsc

POS_IOU_THR = 0.5
NEG_IOU_THR = 0.4

NUM_CORES = 2
NUM_SUBCORES = 16
LANES = 16
NW = NUM_CORES * NUM_SUBCORES


CHUNKS_PER_STEP = 2
GT_UNROLL = 4


@functools.lru_cache(maxsize=None)
def _build_assign(sc_n: int, num_gt: int):
    per_w = sc_n // NW
    chunks = per_w // LANES
    assert per_w % 128 == 0 and chunks % CHUNKS_PER_STEP == 0
    gstride = num_gt * LANES
    mesh = plsc.VectorSubcoreMesh(core_axis_name="c", subcore_axis_name="s",
                                  num_cores=NUM_CORES)

    @functools.partial(
        pl.kernel,
        mesh=mesh,
        out_type=(
            jax.ShapeDtypeStruct((sc_n,), jnp.int32),
            jax.ShapeDtypeStruct((sc_n,), jnp.float32),
            jax.ShapeDtypeStruct((sc_n,), jnp.int32),
        ),
        scratch_types=[
            pltpu.VMEM((4, per_w), jnp.float32),
            pltpu.VMEM((5 * gstride,), jnp.float32),
            pltpu.VMEM((gstride,), jnp.int32),
            pltpu.VMEM((per_w,), jnp.int32),
            pltpu.VMEM((per_w,), jnp.float32),
            pltpu.VMEM((per_w,), jnp.int32),
            pltpu.VMEM((8 * (num_gt // 8 + 1) * 8,), jnp.float32),
        ],
    )
    def assign(coords_hbm, tgt_hbm,
               inds_hbm, maxov_hbm, albl_hbm,
               coords_v, gt_v, lbl_v, inds_v, maxov_v, albl_v, tgt_v):
        wid = lax.axis_index("s") * NUM_CORES + lax.axis_index("c")
        base = wid * per_w
        pltpu.sync_copy(coords_hbm.at[:, pl.ds(base, per_w)], coords_v)
        pltpu.sync_copy(tgt_hbm, tgt_v)

        def build_body(j, carry):
            row = tgt_v[pl.ds(j * 8, LANES)]
            lab = row[4]
            k = lab != -1.0
            gx1 = jnp.where(k, row[0], 0.0)
            gy1 = jnp.where(k, row[1], 0.0)
            gx2 = jnp.where(k, row[2], 0.0)
            gy2 = jnp.where(k, row[3], 0.0)
            ga = (jnp.maximum(gx2 - gx1, 0.0) *
                  jnp.maximum(gy2 - gy1, 0.0))
            j16 = j * LANES
            gt_v[pl.ds(j16, LANES)] = jnp.full((LANES,), gx1)
            gt_v[pl.ds(j16 + gstride, LANES)] = jnp.full((LANES,), gy1)
            gt_v[pl.ds(j16 + 2 * gstride, LANES)] = jnp.full((LANES,), gx2)
            gt_v[pl.ds(j16 + 3 * gstride, LANES)] = jnp.full((LANES,), gy2)
            gt_v[pl.ds(j16 + 4 * gstride, LANES)] = jnp.full((LANES,), ga)
            lbl_v[pl.ds(j16, LANES)] = jnp.full(
                (LANES,), lab).astype(jnp.int32)
            return carry

        lax.fori_loop(0, num_gt, build_body, 0)

        nC = CHUNKS_PER_STEP

        def chunk_body(c, carry):
            offs = [c * (nC * LANES) + k * LANES for k in range(nC)]
            ax1 = [coords_v[0, pl.ds(o, LANES)] for o in offs]
            ay1 = [coords_v[1, pl.ds(o, LANES)] for o in offs]
            ax2 = [coords_v[2, pl.ds(o, LANES)] for o in offs]
            ay2 = [coords_v[3, pl.ds(o, LANES)] for o in offs]
            area1 = [(jnp.maximum(ax2[k] - ax1[k], 0.0) *
                      jnp.maximum(ay2[k] - ay1[k], 0.0)) for k in range(nC)]

            def gt_body(j, bcarry):
                best, bestj, bestl = bcarry
                j16 = j * LANES
                gx1 = gt_v[pl.ds(j16, LANES)]
                gy1 = gt_v[pl.ds(j16 + gstride, LANES)]
                gx2 = gt_v[pl.ds(j16 + 2 * gstride, LANES)]
                gy2 = gt_v[pl.ds(j16 + 3 * gstride, LANES)]
                ga = gt_v[pl.ds(j16 + 4 * gstride, LANES)]
                glb = lbl_v[pl.ds(j16, LANES)]
                jvec = jnp.full((LANES,), j, jnp.int32)
                best, bestj, bestl = list(best), list(bestj), list(bestl)
                for k in range(nC):
                    iw = jnp.maximum(
                        jnp.minimum(ax2[k], gx2) - jnp.maximum(ax1[k], gx1),
                        0.0)
                    ih = jnp.maximum(
                        jnp.minimum(ay2[k], gy2) - jnp.maximum(ay1[k], gy1),
                        0.0)
                    inter = iw * ih
                    union = (area1[k] + ga) - inter
                    iou = inter / jnp.maximum(union, 1e-10)
                    upd = iou > best[k]
                    best[k] = jnp.where(upd, iou, best[k])
                    bestj[k] = jnp.where(upd, jvec, bestj[k])
                    bestl[k] = jnp.where(upd, glb, bestl[k])
                return tuple(best), tuple(bestj), tuple(bestl)

            best0 = tuple(jnp.full((LANES,), -1.0, jnp.float32)
                          for _ in range(nC))
            bestj0 = tuple(jnp.zeros((LANES,), jnp.int32) for _ in range(nC))
            best, bestj, bestl = lax.fori_loop(
                0, num_gt, gt_body, (best0, bestj0, bestj0),
                unroll=GT_UNROLL)

            for k in range(nC):
                pos = best[k] > POS_IOU_THR
                neg = best[k] < NEG_IOU_THR
                inds = jnp.where(
                    pos, bestj[k] + 1,
                    jnp.where(neg, jnp.full((LANES,), 0, jnp.int32),
                              jnp.full((LANES,), -1, jnp.int32)))
                albl = jnp.where(pos, bestl[k],
                                 jnp.full((LANES,), -1, jnp.int32))
                inds_v[pl.ds(offs[k], LANES)] = inds
                maxov_v[pl.ds(offs[k], LANES)] = best[k]
                albl_v[pl.ds(offs[k], LANES)] = albl
            return carry

        lax.fori_loop(0, chunks // nC, chunk_body, 0)
        pltpu.sync_copy(inds_v, inds_hbm.at[pl.ds(base, per_w)])
        pltpu.sync_copy(maxov_v, maxov_hbm.at[pl.ds(base, per_w)])
        pltpu.sync_copy(albl_v, albl_hbm.at[pl.ds(base, per_w)])

    return assign


TC_BLOCK = 2048
SC_N = 4096


@functools.lru_cache(maxsize=None)
def _build_tc_assign(num_anchors: int, sc_n: int, num_gt_pad: int):
    tc_real = num_anchors - sc_n
    nblk = (tc_real + TC_BLOCK - 1) // TC_BLOCK
    col0 = sc_n // TC_BLOCK

    def body(coords_ref, tgt_ref, inds_ref, maxov_ref, albl_ref):
        ax1 = coords_ref[0:1, :]
        ay1 = coords_ref[1:2, :]
        ax2 = coords_ref[2:3, :]
        ay2 = coords_ref[3:4, :]
        area1 = (jnp.maximum(ax2 - ax1, 0.0) *
                 jnp.maximum(ay2 - ay1, 0.0))
        tlab = tgt_ref[:, 4:5]
        keep = tlab != -1.0
        gx1 = jnp.where(keep, tgt_ref[:, 0:1], 0.0)
        gy1 = jnp.where(keep, tgt_ref[:, 1:2], 0.0)
        gx2 = jnp.where(keep, tgt_ref[:, 2:3], 0.0)
        gy2 = jnp.where(keep, tgt_ref[:, 3:4], 0.0)
        ga = (jnp.maximum(gx2 - gx1, 0.0) *
              jnp.maximum(gy2 - gy1, 0.0))
        lbl = tlab.astype(jnp.int32)
        iw = jnp.maximum(jnp.minimum(ax2, gx2) - jnp.maximum(ax1, gx1), 0.0)
        ih = jnp.maximum(jnp.minimum(ay2, gy2) - jnp.maximum(ay1, gy1), 0.0)
        inter = iw * ih
        union = (area1 + ga) - inter
        iou = inter / jnp.maximum(union, 1e-10)
        mx = jnp.max(iou, axis=0, keepdims=True)
        sub = lax.broadcasted_iota(jnp.int32, iou.shape, 0)
        am = jnp.min(jnp.where(iou == mx, sub, num_gt_pad),
                     axis=0, keepdims=True)
        lblv = jnp.max(jnp.where(sub == am, lbl,
                                 jnp.int32(-2147483648)),
                       axis=0, keepdims=True)
        pos = mx > POS_IOU_THR
        neg = mx < NEG_IOU_THR
        inds_ref[...] = jnp.where(
            pos, am + 1,
            jnp.where(neg, jnp.int32(0), jnp.int32(-1)))[0]
        maxov_ref[...] = mx[0]
        albl_ref[...] = jnp.where(pos, lblv, jnp.int32(-1))[0]

    return pl.pallas_call(
        body,
        grid=(nblk,),
        in_specs=[
            pl.BlockSpec((4, TC_BLOCK), lambda i: (0, col0 + i)),
            pl.BlockSpec((num_gt_pad, 8), lambda i: (0, 0)),
        ],
        out_specs=[
            pl.BlockSpec((TC_BLOCK,), lambda i: (i,)),
            pl.BlockSpec((TC_BLOCK,), lambda i: (i,)),
            pl.BlockSpec((TC_BLOCK,), lambda i: (i,)),
        ],
        out_shape=[
            jax.ShapeDtypeStruct((tc_real,), jnp.int32),
            jax.ShapeDtypeStruct((tc_real,), jnp.float32),
            jax.ShapeDtypeStruct((tc_real,), jnp.int32),
        ],
    )


def kernel(bboxes, targets):
    num_anchors = bboxes.shape[0]
    num_gt = targets.shape[0]
    if num_gt == 0:
        return None

    num_gt_pad = (num_gt // 8 + 1) * 8
    tgt_pad = jnp.pad(targets, ((0, num_gt_pad - num_gt), (0, 3)))

    sc_n = min(SC_N, (num_anchors // 4096) * 4096)
    coords = bboxes.T

    parts = []
    if num_anchors - sc_n > 0:
        assign_tc = _build_tc_assign(num_anchors, sc_n, num_gt_pad)
        parts.append(assign_tc(coords, tgt_pad))
    if sc_n > 0:
        assign_sc = _build_assign(sc_n, num_gt)
        parts.insert(0, assign_sc(coords, tgt_pad.reshape(-1)))
    if len(parts) == 1:
        return tuple(parts[0])
    return tuple(jnp.concatenate([a, b]) for a, b in zip(*parts))
